# baseline (speedup 1.0000x reference)
import functools
import math

import jax
import jax.numpy as jnp
from jax import lax
from jax.experimental import pallas as pl
from jax.experimental.pallas import tpu as pltpu
from jax.experimental.pallas import tpu_sc as plsc

D_MODEL = 64
D_PAD = 128
SCALE = math.sqrt(D_MODEL)


@functools.cache
def _make_sc_lookup(batch: int, seq: int):
    info = plsc.get_sparse_core_info()
    nw = info.num_cores * info.num_subcores
    assert batch % nw == 0 and seq % 8 == 0
    bpw = batch // nw
    splits = list(range(0, seq, 128)) + [seq]
    groups = [(splits[i], splits[i + 1] - splits[i])
              for i in range(len(splits) - 1)]

    mesh = plsc.VectorSubcoreMesh(core_axis_name="c", subcore_axis_name="s")

    @functools.partial(
        pl.kernel,
        out_type=jax.ShapeDtypeStruct((batch, seq, D_MODEL), jnp.float32),
        mesh=mesh,
        scratch_types=[
            pltpu.VMEM((bpw * seq,), jnp.int32),
            pltpu.VMEM((2, seq, D_PAD), jnp.float32),
            pltpu.VMEM((2, seq, D_MODEL), jnp.float32),
            pltpu.SemaphoreType.DMA,
            pltpu.SemaphoreType.DMA,
        ],
    )
    def lookup(x_hbm, lut_hbm, out_hbm, idx_v, grows_v, orows_v,
               gsem, wsem):
        wid = lax.axis_index("s") * info.num_cores + lax.axis_index("c")
        b0 = wid * bpw
        pltpu.sync_copy(x_hbm.at[pl.ds(b0 * seq, bpw * seq)], idx_v)

        def fire_gathers(bi, buf):
            base = bi * seq
            for off, length in groups:
                pltpu.async_copy(
                    lut_hbm.at[idx_v.at[pl.ds(
                        pl.multiple_of(base + off, 8), length)]],
                    grows_v.at[buf, pl.ds(off, length)], gsem)

        def wait_gathers():
            for off, length in groups:
                pltpu.make_async_copy(
                    lut_hbm.at[pl.ds(0, length)],
                    grows_v.at[0, pl.ds(off, length)], gsem).wait()

        def fire_write(bi, buf):
            pltpu.async_copy(orows_v.at[buf], out_hbm.at[b0 + bi], wsem)

        def wait_write():
            pltpu.make_async_copy(orows_v.at[0], out_hbm.at[0], wsem).wait()

        fire_gathers(0, 0)

        def row_body(bi, carry):
            buf = bi & 1
            wait_gathers()

            @pl.when(bi + 1 < bpw)
            def _():
                fire_gathers(bi + 1, 1 - buf)

            @pl.when(bi >= 1)
            def _():
                wait_write()

            @plsc.parallel_loop(0, seq, unroll=2)
            def _scale(r):
                for t in range(D_MODEL // 16):
                    v = grows_v[buf, r, pl.ds(t * 16, 16)]
                    orows_v[buf, r, pl.ds(t * 16, 16)] = v * SCALE

            fire_write(bi, buf)
            return carry

        lax.fori_loop(0, bpw, row_body, 0)
        wait_write()

    return lookup


def kernel(x, lut):
    b, s = x.shape
    x1 = x.reshape(-1).astype(jnp.int32)
    lut_p = jnp.pad(lut, ((0, 0), (0, D_PAD - D_MODEL)))
    return _make_sc_lookup(b, s)(x1, lut_p)
